# baseline (speedup 1.0000x reference)
import functools

import jax
import jax.numpy as jnp
from jax.experimental import pallas as pl
from jax.experimental.pallas import tpu as pltpu

_DS = 4
_K = 16
_LAYER_DIMS = [(3, 16, 32), (32, 32, 64), (64, 64, 128), (128, 128, 256)]
_BIG = 3.4e38


_RB = 64


def _layer_body(p_ref, ct_ref, gt_ref, w1gt_ref, b1_ref, w2t_ref, b2_ref,
                w3t_ref, b3_ref, out_ref, d2_ref, *, nh, f):
    p = p_ref[0]
    c = ct_ref[0]
    GT4 = gt_ref[0]
    w1gt = w1gt_ref[...]
    n = p.shape[0]
    mt = c.shape[1]
    nl = n // nh
    rb = min(_RB, n)

    d2_ref[...] = ((p[:, 0:1] - c[0:1, :]) ** 2
                   + (p[:, 1:2] - c[1:2, :]) ** 2
                   + (p[:, 2:3] - c[2:3, :]) ** 2)
    iota_rb = jax.lax.broadcasted_iota(jnp.int32, (rb, mt), 0)
    iota_h = jax.lax.broadcasted_iota(jnp.int32, (nh, mt), 0)
    iota_l = jax.lax.broadcasted_iota(jnp.int32, (nl, mt), 0)

    cb = jnp.dot(w1gt[:, 0:3], c, preferred_element_type=jnp.float32)
    t1 = b1_ref[...] - cb

    def _scan_chunk(h, carry):
        bv, bi, jprev = carry
        blk = d2_ref[pl.ds(h * rb, rb), :]
        idxv = iota_rb + h * rb
        blk = jnp.where(idxv == jprev, _BIG, blk)
        d2_ref[pl.ds(h * rb, rb), :] = blk
        lt = blk < bv
        return (jnp.where(lt, blk, bv), jnp.where(lt, idxv, bi), jprev)

    acc = None
    jidx = jnp.full((1, mt), -1, jnp.int32)
    for _ in range(_K):
        bv0 = jnp.full((rb, mt), _BIG, jnp.float32)
        bi0 = jnp.full((rb, mt), n, jnp.int32)
        bv, bi, _ = jax.lax.fori_loop(0, n // rb, _scan_chunk,
                                      (bv0, bi0, jidx))
        rmin = jnp.min(bv, axis=0, keepdims=True)
        jidx = jnp.min(jnp.where(bv == rmin, bi, n), axis=0, keepdims=True)
        lo_f = (iota_l == jidx % nl).astype(jnp.float32)
        D = jax.lax.dot_general(GT4, lo_f,
                                (((1,), (0,)), ((), ())),
                                precision=jax.lax.Precision.HIGHEST,
                                preferred_element_type=jnp.float32)
        if nh > 1:
            hi_f = (iota_h == jidx // nl).astype(jnp.float32)
            g = jnp.sum(D.reshape(f, nh, mt) * hi_f[None], axis=1)
        else:
            g = D
        h = jnp.maximum(
            jnp.dot(w1gt, g, preferred_element_type=jnp.float32) + t1, 0.0)
        h = jnp.maximum(
            jnp.dot(w2t_ref[...], h, preferred_element_type=jnp.float32)
            + b2_ref[...], 0.0)
        h = jnp.dot(w3t_ref[...], h, preferred_element_type=jnp.float32)
        acc = h if acc is None else jnp.maximum(acc, h)
    out_ref[0] = acc + b3_ref[...]


def _layer(posl, ptt, featt, params, li, mt):
    B, N, _ = posl.shape
    M = N // _DS
    C = featt.shape[1]
    cin, hid, cout = _LAYER_DIMS[li]
    W1 = params['l%d_W1' % li]
    W1gt = jnp.concatenate([W1[C:], W1[:C]], axis=0).T
    GT = jnp.concatenate([ptt, featt], axis=1)
    NH = N // 128 if N >= 1024 else 1
    NL = N // NH
    GT4 = GT.reshape(B, (3 + C) * NH, NL)
    CTT = ptt[:, :, :M]
    b1 = params['l%d_b1' % li].reshape(-1, 1)
    b2 = params['l%d_b2' % li].reshape(-1, 1)
    b3 = params['l%d_b3' % li].reshape(-1, 1)
    W2T = params['l%d_W2' % li].T
    W3T = params['l%d_W3' % li].T
    F = 3 + C

    grid = (B, M // mt)
    out = pl.pallas_call(
        functools.partial(_layer_body, nh=NH, f=F),
        grid=grid,
        in_specs=[
            pl.BlockSpec((1, N, 3), lambda b, m: (b, 0, 0)),
            pl.BlockSpec((1, 3, mt), lambda b, m: (b, 0, m)),
            pl.BlockSpec((1, F * NH, NL), lambda b, m: (b, 0, 0)),
            pl.BlockSpec((hid, F), lambda b, m: (0, 0)),
            pl.BlockSpec((hid, 1), lambda b, m: (0, 0)),
            pl.BlockSpec((2 * hid, hid), lambda b, m: (0, 0)),
            pl.BlockSpec((2 * hid, 1), lambda b, m: (0, 0)),
            pl.BlockSpec((cout, 2 * hid), lambda b, m: (0, 0)),
            pl.BlockSpec((cout, 1), lambda b, m: (0, 0)),
        ],
        out_specs=pl.BlockSpec((1, cout, mt), lambda b, m: (b, 0, m)),
        out_shape=jax.ShapeDtypeStruct((B, cout, M), jnp.float32),
        scratch_shapes=[pltpu.VMEM((N, mt), jnp.float32)],
    )(posl, CTT, GT4, W1gt, b1, W2T, b2, W3T, b3)
    return out


def _head_body(f_ref, w1_ref, b1_ref, w2_ref, b2_ref, out_ref):
    f = jnp.mean(f_ref[...], axis=2)
    h = jnp.maximum(
        jnp.dot(f, w1_ref[...], preferred_element_type=jnp.float32)
        + b1_ref[...], 0.0)
    o = jnp.dot(h, w2_ref[...], preferred_element_type=jnp.float32) \
        + b2_ref[...]
    out_ref[...] = jax.nn.sigmoid(o)


def kernel(x, params):
    B, N0, _ = x.shape
    featt = jnp.transpose(x[:, :, 3:6], (0, 2, 1))
    N = N0
    tiles = [128, 256, 64, 16]
    for li in range(4):
        posl = x[:, :N, :3]
        ptt = jnp.transpose(posl, (0, 2, 1))
        featt = _layer(posl, ptt, featt, params, li,
                       min(tiles[li], N // _DS))
        N = N // _DS

    out = pl.pallas_call(
        _head_body,
        in_specs=[pl.BlockSpec(featt.shape, lambda: (0, 0, 0)),
                  pl.BlockSpec((256, 64), lambda: (0, 0)),
                  pl.BlockSpec((1, 64), lambda: (0, 0)),
                  pl.BlockSpec((64, 40), lambda: (0, 0)),
                  pl.BlockSpec((1, 40), lambda: (0, 0))],
        out_specs=pl.BlockSpec((B, 40), lambda: (0, 0)),
        out_shape=jax.ShapeDtypeStruct((B, 40), jnp.float32),
    )(featt, params['cls_W1'], params['cls_b1'].reshape(1, -1),
      params['cls_W2'], params['cls_b2'].reshape(1, -1))
    return out


# speedup vs baseline: 1.9974x; 1.9974x over previous
import functools

import jax
import jax.numpy as jnp
from jax.experimental import pallas as pl
from jax.experimental.pallas import tpu as pltpu

_DS = 4
_K = 16
_LAYER_DIMS = [(3, 16, 32), (32, 32, 64), (64, 64, 128), (128, 128, 256)]
_BIG = 3.4e38


def _layer_body(p_ref, ct_ref, gt_ref, w1gt_ref, b1_ref, w2t_ref, b2_ref,
                w3t_ref, b3_ref, out_ref, *, nh, f):
    p = p_ref[0]
    c = ct_ref[0]
    GT4 = gt_ref[0]
    w1gt = w1gt_ref[...]
    n = p.shape[0]
    mt = c.shape[1]
    nl = n // nh

    d2 = ((p[:, 0:1] - c[0:1, :]) ** 2
          + (p[:, 1:2] - c[1:2, :]) ** 2
          + (p[:, 2:3] - c[2:3, :]) ** 2)
    iota = jax.lax.broadcasted_iota(jnp.int32, (n, mt), 0)
    iota_h = jax.lax.broadcasted_iota(jnp.int32, (nh, mt), 0)
    iota_l = jax.lax.broadcasted_iota(jnp.int32, (nl, mt), 0)

    cb = jnp.dot(w1gt[:, 0:3], c, preferred_element_type=jnp.float32)
    t1 = b1_ref[...] - cb

    acc = None
    for _ in range(_K):
        jidx = jnp.argmin(d2, axis=0, keepdims=True)
        d2 = jnp.where(iota == jidx, _BIG, d2)
        lo_f = (iota_l == jidx % nl).astype(jnp.float32)
        D = jax.lax.dot_general(GT4, lo_f,
                                (((1,), (0,)), ((), ())),
                                precision=jax.lax.Precision.HIGHEST,
                                preferred_element_type=jnp.float32)
        if nh > 1:
            hi_f = (iota_h == jidx // nl).astype(jnp.float32)
            g = jnp.sum(D.reshape(f, nh, mt) * hi_f[None], axis=1)
        else:
            g = D
        h = jnp.maximum(
            jnp.dot(w1gt, g, preferred_element_type=jnp.float32) + t1, 0.0)
        h = jnp.maximum(
            jnp.dot(w2t_ref[...], h, preferred_element_type=jnp.float32)
            + b2_ref[...], 0.0)
        h = jnp.dot(w3t_ref[...], h, preferred_element_type=jnp.float32)
        acc = h if acc is None else jnp.maximum(acc, h)
    out_ref[0] = acc + b3_ref[...]


def _layer(posl, ptt, featt, params, li, mt):
    B, N, _ = posl.shape
    M = N // _DS
    C = featt.shape[1]
    cin, hid, cout = _LAYER_DIMS[li]
    W1 = params['l%d_W1' % li]
    W1gt = jnp.concatenate([W1[C:], W1[:C]], axis=0).T
    GT = jnp.concatenate([ptt, featt], axis=1)
    NH = N // 128 if N >= 1024 else 1
    NL = N // NH
    GT4 = GT.reshape(B, (3 + C) * NH, NL)
    CTT = ptt[:, :, :M]
    b1 = params['l%d_b1' % li].reshape(-1, 1)
    b2 = params['l%d_b2' % li].reshape(-1, 1)
    b3 = params['l%d_b3' % li].reshape(-1, 1)
    W2T = params['l%d_W2' % li].T
    W3T = params['l%d_W3' % li].T
    F = 3 + C

    grid = (B, M // mt)
    out = pl.pallas_call(
        functools.partial(_layer_body, nh=NH, f=F),
        grid=grid,
        in_specs=[
            pl.BlockSpec((1, N, 3), lambda b, m: (b, 0, 0)),
            pl.BlockSpec((1, 3, mt), lambda b, m: (b, 0, m)),
            pl.BlockSpec((1, F * NH, NL), lambda b, m: (b, 0, 0)),
            pl.BlockSpec((hid, F), lambda b, m: (0, 0)),
            pl.BlockSpec((hid, 1), lambda b, m: (0, 0)),
            pl.BlockSpec((2 * hid, hid), lambda b, m: (0, 0)),
            pl.BlockSpec((2 * hid, 1), lambda b, m: (0, 0)),
            pl.BlockSpec((cout, 2 * hid), lambda b, m: (0, 0)),
            pl.BlockSpec((cout, 1), lambda b, m: (0, 0)),
        ],
        out_specs=pl.BlockSpec((1, cout, mt), lambda b, m: (b, 0, m)),
        out_shape=jax.ShapeDtypeStruct((B, cout, M), jnp.float32),
    )(posl, CTT, GT4, W1gt, b1, W2T, b2, W3T, b3)
    return out


def _head_body(f_ref, w1_ref, b1_ref, w2_ref, b2_ref, out_ref):
    f = jnp.mean(f_ref[...], axis=2)
    h = jnp.maximum(
        jnp.dot(f, w1_ref[...], preferred_element_type=jnp.float32)
        + b1_ref[...], 0.0)
    o = jnp.dot(h, w2_ref[...], preferred_element_type=jnp.float32) \
        + b2_ref[...]
    out_ref[...] = jax.nn.sigmoid(o)


def kernel(x, params):
    B, N0, _ = x.shape
    featt = jnp.transpose(x[:, :, 3:6], (0, 2, 1))
    N = N0
    tiles = [256, 256, 64, 16]
    for li in range(4):
        posl = x[:, :N, :3]
        ptt = jnp.transpose(posl, (0, 2, 1))
        featt = _layer(posl, ptt, featt, params, li,
                       min(tiles[li], N // _DS))
        N = N // _DS

    out = pl.pallas_call(
        _head_body,
        in_specs=[pl.BlockSpec(featt.shape, lambda: (0, 0, 0)),
                  pl.BlockSpec((256, 64), lambda: (0, 0)),
                  pl.BlockSpec((1, 64), lambda: (0, 0)),
                  pl.BlockSpec((64, 40), lambda: (0, 0)),
                  pl.BlockSpec((1, 40), lambda: (0, 0))],
        out_specs=pl.BlockSpec((B, 40), lambda: (0, 0)),
        out_shape=jax.ShapeDtypeStruct((B, 40), jnp.float32),
    )(featt, params['cls_W1'], params['cls_b1'].reshape(1, -1),
      params['cls_W2'], params['cls_b2'].reshape(1, -1))
    return out
